# baseline (speedup 1.0000x reference)
import jax
import jax.numpy as jnp
from jax import lax
from jax.experimental import pallas as pl
from jax.experimental.pallas import tpu as pltpu
from jax.experimental.pallas import tpu_sc as plsc

_PLACEHOLDER = 9
_LR_MUL = 0.1

_NC = 2
_NS = 16
_NW = _NC * _NS


def _mlp_body(tok_ref, face_ref, w0_ref, b0_ref, w1_ref, b1_ref, cm_ref, cs_ref,
              tie0_ref, tie1_ref, pos_ref):
    x = face_ref[...]
    nrm = jnp.sqrt(jnp.sum(x * x, axis=1, keepdims=True))
    x = x / jnp.maximum(nrm, 1e-12)
    h = lax.dot_general(x, w0_ref[...], (((1,), (1,)), ((), ())),
                        precision=lax.Precision.HIGHEST,
                        preferred_element_type=jnp.float32)
    h = h * _LR_MUL + b0_ref[...] * _LR_MUL
    h = jnp.where(h >= 0, h, 0.2 * h)
    r = lax.dot_general(h, w1_ref[...], (((1,), (1,)), ((), ())),
                        precision=lax.Precision.HIGHEST,
                        preferred_element_type=jnp.float32)
    r = r * _LR_MUL + b1_ref[...] * _LR_MUL
    r = jnp.where(r >= 0, r, 0.2 * r)
    d = cm_ref.shape[1]
    tie0_ref[...] = cm_ref[0:1, :] + r[:, :d] * cs_ref[0:1, :]
    tie1_ref[...] = cm_ref[1:2, :] + r[:, d:] * cs_ref[1:2, :]
    tok = tok_ref[...]
    bm, n_seq = tok.shape
    iota = lax.broadcasted_iota(jnp.int32, (bm, n_seq), 1)
    pos_ref[...] = jnp.min(jnp.where(tok == _PLACEHOLDER, iota, n_seq + 1),
                           axis=1, keepdims=True)


def _mlp(tokenized_text, face_img_embeddings, W0, b0, W1, b1,
         celeb_mean, celeb_std, batch, n_seq, token_dim):
    dim_out = W0.shape[0]
    vit_dim = face_img_embeddings.shape[1]
    bmlp = 256
    return pl.pallas_call(
        _mlp_body,
        grid=(batch // bmlp,),
        in_specs=[
            pl.BlockSpec((bmlp, n_seq), lambda i: (i, 0)),
            pl.BlockSpec((bmlp, vit_dim), lambda i: (i, 0)),
            pl.BlockSpec((dim_out, vit_dim), lambda i: (0, 0)),
            pl.BlockSpec((1, dim_out), lambda i: (0, 0)),
            pl.BlockSpec((dim_out, dim_out), lambda i: (0, 0)),
            pl.BlockSpec((1, dim_out), lambda i: (0, 0)),
            pl.BlockSpec((2, token_dim), lambda i: (0, 0)),
            pl.BlockSpec((2, token_dim), lambda i: (0, 0)),
        ],
        out_specs=(
            pl.BlockSpec((bmlp, token_dim), lambda i: (i, 0)),
            pl.BlockSpec((bmlp, token_dim), lambda i: (i, 0)),
            pl.BlockSpec((bmlp, 1), lambda i: (i, 0)),
        ),
        out_shape=(
            jax.ShapeDtypeStruct((batch, token_dim), jnp.float32),
            jax.ShapeDtypeStruct((batch, token_dim), jnp.float32),
            jax.ShapeDtypeStruct((batch, 1), jnp.int32),
        ),
    )(tokenized_text, face_img_embeddings, W0, b0.reshape(1, dim_out), W1,
      b1.reshape(1, dim_out), celeb_mean, celeb_std)


def _make_sc_copy_scatter(batch, n_seq, token_dim):
    rows = batch * n_seq
    b_per_w = batch // _NW
    slab = b_per_w * n_seq
    mesh = plsc.VectorSubcoreMesh(core_axis_name="c", subcore_axis_name="s")

    chunk = 56
    nchunks = slab // chunk
    nbuf = 2
    pre = 1

    def body(emb_hbm, tie0_hbm, tie1_hbm, pos_hbm, out_hbm,
             pos_v, i00, i01, i10, i11, sem, bufs, gsems, wsems):
        wid = lax.axis_index("s") * _NC + lax.axis_index("c")
        base_b = wid * b_per_w
        base_row = wid * slab

        def gather(k, b):
            return pltpu.async_copy(
                emb_hbm.at[pl.ds(base_row + k * chunk, chunk)],
                bufs[b], gsems[b])

        def write(k, b):
            return pltpu.async_copy(
                bufs[b], out_hbm.at[pl.ds(base_row + k * chunk, chunk)],
                wsems[b])

        gh = [None] * nbuf
        wh = [None] * nbuf
        for k in range(pre):
            gh[k] = gather(k, k)
        for k in range(nchunks):
            b = k % nbuf
            gk = k + pre
            if gk < nchunks:
                gb = gk % nbuf
                if wh[gb] is not None:
                    wh[gb].wait()
                    wh[gb] = None
                gh[gb] = gather(gk, gb)
            gh[b].wait()
            wh[b] = write(k, b)
        for b in range(nbuf):
            if wh[b] is not None:
                wh[b].wait()

        pltpu.sync_copy(pos_hbm.at[pl.ds(base_b, b_per_w)], pos_v)
        iot = lax.iota(jnp.int32, 16)
        p0 = pos_v[pl.ds(0, 16)]
        r0 = (base_b + iot) * n_seq + p0
        i00[...] = r0
        i10[...] = r0 + 1
        p1 = pos_v[pl.ds(16, 16)]
        r1 = (base_b + 16 + iot) * n_seq + p1
        i01[...] = r1
        i11[...] = r1 + 1

        pltpu.sync_copy(tie0_hbm.at[pl.ds(base_b, 16)], bufs[0].at[pl.ds(0, 16)])
        pltpu.sync_copy(tie0_hbm.at[pl.ds(base_b + 16, 16)],
                        bufs[0].at[pl.ds(16, 16)])
        pltpu.sync_copy(tie1_hbm.at[pl.ds(base_b, 16)], bufs[1].at[pl.ds(0, 16)])
        pltpu.sync_copy(tie1_hbm.at[pl.ds(base_b + 16, 16)],
                        bufs[1].at[pl.ds(16, 16)])
        pltpu.async_copy(bufs[0].at[pl.ds(0, 16)], out_hbm.at[i00], sem).wait()
        pltpu.async_copy(bufs[0].at[pl.ds(16, 16)], out_hbm.at[i01], sem).wait()
        pltpu.async_copy(bufs[1].at[pl.ds(0, 16)], out_hbm.at[i10], sem).wait()
        pltpu.async_copy(bufs[1].at[pl.ds(16, 16)], out_hbm.at[i11], sem).wait()

    return pl.kernel(
        body,
        out_type=jax.ShapeDtypeStruct((rows, token_dim), jnp.float32),
        mesh=mesh,
        scratch_types=[
            pltpu.VMEM((b_per_w,), jnp.int32),
            pltpu.VMEM((16,), jnp.int32),
            pltpu.VMEM((16,), jnp.int32),
            pltpu.VMEM((16,), jnp.int32),
            pltpu.VMEM((16,), jnp.int32),
            pltpu.SemaphoreType.DMA,
            [pltpu.VMEM((chunk, token_dim), jnp.float32) for _ in range(nbuf)],
            [pltpu.SemaphoreType.DMA for _ in range(nbuf)],
            [pltpu.SemaphoreType.DMA for _ in range(nbuf)],
        ],
    )


def kernel(tokenized_text, embedded_text, face_img_embeddings,
           W0, b0, W1, b1, celeb_mean, celeb_std):
    batch, n_seq, token_dim = embedded_text.shape

    tie0, tie1, pos2d = _mlp(tokenized_text, face_img_embeddings,
                             W0, b0, W1, b1, celeb_mean, celeb_std,
                             batch, n_seq, token_dim)
    pos = pos2d.reshape(batch)
    emb2d = embedded_text.reshape(batch * n_seq, token_dim)
    sc = _make_sc_copy_scatter(batch, n_seq, token_dim)
    out2d = sc(emb2d, tie0, tie1, pos)
    return out2d.reshape(batch, n_seq, token_dim)


# speedup vs baseline: 1.5973x; 1.5973x over previous
import jax
import jax.numpy as jnp
from jax import lax
from jax.experimental import pallas as pl
from jax.experimental.pallas import tpu as pltpu

_PLACEHOLDER = 9
_LR_MUL = 0.1


def _mlp_body(face_ref, w0_ref, b0_ref, w1_ref, b1_ref, cm_ref, cs_ref,
              tie0_ref, tie1_ref):
    x = face_ref[...]
    nrm = jnp.sqrt(jnp.sum(x * x, axis=1, keepdims=True))
    x = x / jnp.maximum(nrm, 1e-12)
    h = lax.dot_general(x, w0_ref[...], (((1,), (1,)), ((), ())),
                        preferred_element_type=jnp.float32)
    h = h * _LR_MUL + b0_ref[...] * _LR_MUL
    h = jnp.where(h >= 0, h, 0.2 * h)
    r = lax.dot_general(h, w1_ref[...], (((1,), (1,)), ((), ())),
                        preferred_element_type=jnp.float32)
    r = r * _LR_MUL + b1_ref[...] * _LR_MUL
    r = jnp.where(r >= 0, r, 0.2 * r)
    d = cm_ref.shape[1]
    tie0_ref[...] = cm_ref[0:1, :] + r[:, :d] * cs_ref[0:1, :]
    tie1_ref[...] = cm_ref[1:2, :] + r[:, d:] * cs_ref[1:2, :]


def _mlp(face_img_embeddings, W0, b0, W1, b1, celeb_mean, celeb_std,
         batch, token_dim):
    dim_out = W0.shape[0]
    vit_dim = face_img_embeddings.shape[1]
    bmlp = 256
    return pl.pallas_call(
        _mlp_body,
        grid=(batch // bmlp,),
        in_specs=[
            pl.BlockSpec((bmlp, vit_dim), lambda i: (i, 0)),
            pl.BlockSpec((dim_out, vit_dim), lambda i: (0, 0)),
            pl.BlockSpec((1, dim_out), lambda i: (0, 0)),
            pl.BlockSpec((dim_out, dim_out), lambda i: (0, 0)),
            pl.BlockSpec((1, dim_out), lambda i: (0, 0)),
            pl.BlockSpec((2, token_dim), lambda i: (0, 0)),
            pl.BlockSpec((2, token_dim), lambda i: (0, 0)),
        ],
        out_specs=(
            pl.BlockSpec((bmlp, token_dim), lambda i: (i, 0)),
            pl.BlockSpec((bmlp, token_dim), lambda i: (i, 0)),
        ),
        out_shape=(
            jax.ShapeDtypeStruct((batch, token_dim), jnp.float32),
            jax.ShapeDtypeStruct((batch, token_dim), jnp.float32),
        ),
    )(face_img_embeddings, W0, b0.reshape(1, dim_out), W1,
      b1.reshape(1, dim_out), celeb_mean, celeb_std)


def _merge_body(tok_ref, emb_ref, tie0_ref, tie1_ref, out_ref):
    bm, n_seq = tok_ref.shape
    tok = tok_ref[...]
    iota = lax.broadcasted_iota(jnp.int32, (bm, n_seq), 1)
    posk = jnp.min(jnp.where(tok == _PLACEHOLDER, iota, n_seq + 1),
                   axis=1, keepdims=True)
    token_dim = tie0_ref.shape[1]
    shape3 = (bm, n_seq, token_dim)
    m0 = lax.broadcast_in_dim((iota == posk).astype(jnp.float32), shape3, (0, 1))
    m1 = lax.broadcast_in_dim((iota == posk + 1).astype(jnp.float32),
                              shape3, (0, 1))
    t0 = lax.broadcast_in_dim(tie0_ref[...], shape3, (0, 2))
    t1 = lax.broadcast_in_dim(tie1_ref[...], shape3, (0, 2))
    out_ref[...] = (emb_ref[...] * (1.0 - m0 - m1)) + m0 * t0 + m1 * t1


def kernel(tokenized_text, embedded_text, face_img_embeddings,
           W0, b0, W1, b1, celeb_mean, celeb_std):
    batch, n_seq, token_dim = embedded_text.shape

    tie0, tie1 = _mlp(face_img_embeddings, W0, b0, W1, b1,
                      celeb_mean, celeb_std, batch, token_dim)

    bm = 32
    out = pl.pallas_call(
        _merge_body,
        grid=(batch // bm,),
        in_specs=[
            pl.BlockSpec((bm, n_seq), lambda i: (i, 0)),
            pl.BlockSpec((bm, n_seq, token_dim), lambda i: (i, 0, 0)),
            pl.BlockSpec((bm, token_dim), lambda i: (i, 0)),
            pl.BlockSpec((bm, token_dim), lambda i: (i, 0)),
        ],
        out_specs=pl.BlockSpec((bm, n_seq, token_dim), lambda i: (i, 0, 0)),
        out_shape=jax.ShapeDtypeStruct((batch, n_seq, token_dim), jnp.float32),
        compiler_params=pltpu.CompilerParams(
            dimension_semantics=("arbitrary",),
        ),
    )(tokenized_text, embedded_text, tie0, tie1)
    return out
